# Initial kernel scaffold; baseline (speedup 1.0000x reference)
import jax
import jax.numpy as jnp
from jax.experimental import pallas as pl


def kernel(feature, ROIs):
    raise NotImplementedError("write your pallas kernel here")


# speedup vs baseline: 18.3118x; 18.3118x over previous
import functools

import jax
import jax.numpy as jnp
from jax import lax
from jax.experimental import pallas as pl
from jax.experimental.pallas import tpu as pltpu
from jax.experimental.pallas import tpu_sc as plsc

CH, CW = 16, 16


def kernel(feature, ROIs):
    B, H, W, C = feature.shape
    N = ROIs.shape[0]
    NW = 32
    per_w = (N + NW - 1) // NW
    R = CH * CW

    feat_tbl = feature.reshape(B * H * W, C)
    rois_flat = jnp.pad(ROIs.reshape(N, 6), ((0, 1), (0, 2))).reshape(-1)

    mesh = plsc.VectorSubcoreMesh(
        core_axis_name="c", subcore_axis_name="s", num_cores=2, num_subcores=16
    )

    @functools.partial(
        pl.kernel,
        out_type=jax.ShapeDtypeStruct((N * R, C), jnp.float32),
        mesh=mesh,
        compiler_params=pltpu.CompilerParams(use_tc_tiling_on_sc=False),
        scratch_types=[
            pltpu.VMEM(((N + 1) * 8,), jnp.int32),
            pltpu.VMEM((R,), jnp.int32),
            pltpu.VMEM((16,), jnp.int32),
            pltpu.VMEM((R, C), jnp.float32),
            pltpu.SemaphoreType.DMA,
        ],
    )
    def _crop(feat_hbm, rois_hbm, out_hbm, rois_v, idx_v, civ_v, rows_v, sem):
        wid = lax.axis_index("s") * 2 + lax.axis_index("c")
        pltpu.sync_copy(rois_hbm, rois_v)
        start = wid * per_w
        cnt = jnp.maximum(0, jnp.minimum(per_w, N - start))

        lane = lax.iota(jnp.int32, 16)
        zeros16 = jnp.zeros((16,), jnp.float32)

        def one_roi(t, carry):
            n = start + t
            rec = rois_v[pl.ds(n * 8, 16)]
            b = rec[0]
            y1 = rec[1]
            x1 = rec[2]
            cols = x1 + lane
            civ_v[...] = jnp.where((cols < 0) | (cols >= W), 1, 0)
            cc = jnp.clip(cols, 0, W - 1)
            interior = (y1 >= 0) & (y1 <= H - CH) & (x1 >= 0) & (x1 <= W - CW)
            for i in range(CH):
                y = y1 + i
                rc = jnp.clip(y, 0, H - 1)
                base = (b * H + rc) * W
                idx_v[pl.ds(i * CW, CW)] = base + cc
            pltpu.async_copy(feat_hbm.at[idx_v], rows_v, sem).wait()

            @pl.when(jnp.logical_not(interior))
            def _zero_oob():
                civ = civ_v[...]
                for i in range(CH):
                    y = y1 + i
                    rinv = ((y < 0) | (y >= H)).astype(jnp.int32)
                    for j in range(CW):
                        @pl.when((rinv + civ[j]) > 0)
                        def _z(i=i, j=j):
                            for v in range(C // 16):
                                rows_v[i * CW + j, pl.ds(v * 16, 16)] = zeros16

            pltpu.sync_copy(rows_v, out_hbm.at[pl.ds(n * R, R)])
            return carry

        lax.fori_loop(0, cnt, one_roi, 0)

    out = _crop(feat_tbl, rois_flat)
    return out.reshape(N, CH, CW, C)


# speedup vs baseline: 18.8913x; 1.0316x over previous
import functools

import jax
import jax.numpy as jnp
from jax import lax
from jax.experimental import pallas as pl
from jax.experimental.pallas import tpu as pltpu
from jax.experimental.pallas import tpu_sc as plsc

CH, CW = 16, 16
K = 2


def kernel(feature, ROIs):
    B, H, W, C = feature.shape
    N = ROIs.shape[0]
    NW = 32
    per_w = (N + NW - 1) // NW
    R = CH * CW

    feat_tbl = feature.reshape(B * H * W, C)
    rois_flat = jnp.pad(
        ROIs.reshape(N, 6), ((0, NW * per_w + 1 - N), (0, 2))
    ).reshape(-1)

    mesh = plsc.VectorSubcoreMesh(
        core_axis_name="c", subcore_axis_name="s", num_cores=2, num_subcores=16
    )

    @functools.partial(
        pl.kernel,
        out_type=jax.ShapeDtypeStruct((N * R, C), jnp.float32),
        mesh=mesh,
        compiler_params=pltpu.CompilerParams(use_tc_tiling_on_sc=False),
        scratch_types=[
            pltpu.VMEM(((per_w + 1) * 8,), jnp.int32),
            pltpu.VMEM((2, K * R), jnp.int32),
            pltpu.VMEM((16,), jnp.int32),
            pltpu.VMEM((2, K * R, C), jnp.float32),
            pltpu.SemaphoreType.DMA,
            pltpu.SemaphoreType.DMA,
        ],
    )
    def _crop(feat_hbm, rois_hbm, out_hbm, rois_v, idx_v, civ_v, rows_v,
              sem0, sem1):
        wid = lax.axis_index("s") * 2 + lax.axis_index("c")
        start = wid * per_w
        pltpu.sync_copy(rois_hbm.at[pl.ds(start * 8, (per_w + 1) * 8)], rois_v)
        cnt = jnp.maximum(0, jnp.minimum(per_w, N - start))
        nchunks = cnt // K
        sems = (sem0, sem1)

        lane = lax.iota(jnp.int32, 16)
        zeros16 = jnp.zeros((16,), jnp.float32)

        def issue(c, buf):
            for k in range(K):
                t = c * K + k
                rec = rois_v[pl.ds(t * 8, 16)]
                b = rec[0]
                y1 = rec[1]
                x1 = rec[2]
                cc = jnp.clip(x1 + lane, 0, W - 1)
                for i in range(CH):
                    rc = jnp.clip(y1 + i, 0, H - 1)
                    base = (b * H + rc) * W
                    idx_v[buf, pl.ds(k * R + i * CW, CW)] = base + cc
            pltpu.async_copy(
                feat_hbm.at[idx_v.at[buf]], rows_v.at[buf], sems[buf]
            )

        def drain(c, buf):
            pltpu.make_async_copy(
                feat_hbm.at[idx_v.at[buf]], rows_v.at[buf], sems[buf]
            ).wait()
            for k in range(K):
                t = c * K + k
                rec = rois_v[pl.ds(t * 8, 16)]
                y1 = rec[1]
                x1 = rec[2]
                cols = x1 + lane
                civ_v[...] = jnp.where((cols < 0) | (cols >= W), 1, 0)
                interior = (
                    (y1 >= 0) & (y1 <= H - CH) & (x1 >= 0) & (x1 <= W - CW)
                )

                @pl.when(jnp.logical_not(interior))
                def _zero_oob(k=k, y1=y1):
                    civ = civ_v[...]

                    def zrow(i, cr):
                        y = y1 + i
                        rinv = ((y < 0) | (y >= H)).astype(jnp.int32)
                        g0 = k * R + i * CW
                        for j in range(CW):
                            @pl.when((rinv + civ[j]) > 0)
                            def _z(j=j, g0=g0):
                                for v in range(C // 16):
                                    rows_v[buf, g0 + j,
                                           pl.ds(v * 16, 16)] = zeros16
                        return cr

                    lax.fori_loop(0, CH, zrow, 0)

            pltpu.sync_copy(
                rows_v.at[buf],
                out_hbm.at[pl.ds((start + c * K) * R, K * R)],
            )

        @pl.when(nchunks > 0)
        def _pipeline():
            issue(0, 0)

            def outer(p, carry):
                c0 = 2 * p

                @pl.when(c0 + 1 < nchunks)
                def _i1():
                    issue(c0 + 1, 1)

                drain(c0, 0)

                @pl.when(c0 + 2 < nchunks)
                def _i2():
                    issue(c0 + 2, 0)

                @pl.when(c0 + 1 < nchunks)
                def _d1():
                    drain(c0 + 1, 1)

                return carry

            lax.fori_loop(0, (nchunks + 1) // 2, outer, 0)

    out = _crop(feat_tbl, rois_flat)
    return out.reshape(N, CH, CW, C)
